# baseline (speedup 1.0000x reference)
import jax
import jax.numpy as jnp
from jax.experimental import pallas as pl
from jax.experimental.pallas import tpu as pltpu

_B, _S, _D = 4, 8192, 768
_E, _TOPK = 8, 2
_Z_LOSS_COEFF = 1e-05
_BIAS_UPDATE_SPEED = 0.001

_BT = 8192
_N = _B * _S
_G = _N // _BT
_LANES = 128


def _router_kernel(x_ref, w_ref, b_ref, idx_ref, wts_ref, st_ref):
    x = x_ref[...]
    wt = w_ref[...].T
    lt = jax.lax.dot_general(
        wt, x, (((1,), (1,)), ((), ())),
        preferred_element_type=jnp.float32)
    biased = lt + b_ref[...]

    row = jax.lax.broadcasted_iota(jnp.int32, (_E, _BT), 0)
    i1 = jnp.argmax(biased, axis=0).astype(jnp.int32)
    eq1 = row == i1[None, :]
    masked = jnp.where(eq1, -jnp.inf, biased)
    i2 = jnp.argmax(masked, axis=0).astype(jnp.int32)
    eq2 = row == i2[None, :]

    l1 = jnp.sum(jnp.where(eq1, lt, 0.0), axis=0)
    l2 = jnp.sum(jnp.where(eq2, lt, 0.0), axis=0)
    t = jnp.exp(l2 - l1)
    w1v = 1.0 / (1.0 + t)

    mx = jnp.max(lt, axis=0)
    lz = mx + jnp.log(jnp.sum(jnp.exp(lt - mx[None, :]), axis=0))
    zsq = lz * lz

    c = eq1.astype(jnp.float32) + eq2.astype(jnp.float32)
    cpart = c[:, 0:_LANES]
    zpart = zsq[0:_LANES]
    for j in range(1, _BT // _LANES):
        cpart = cpart + c[:, j * _LANES:(j + 1) * _LANES]
        zpart = zpart + zsq[j * _LANES:(j + 1) * _LANES]

    idx_ref[0:1, :] = i1[None]
    idx_ref[1:2, :] = i2[None]
    wts_ref[0:1, :] = w1v[None]
    wts_ref[1:2, :] = (1.0 - w1v)[None]
    st_ref[0, 0:_E, :] = cpart
    st_ref[0, _E:_E + 1, :] = zpart[None]


@jax.jit
def kernel(x, W, expert_bias, expert_counts, total_tokens):
    xf = x.reshape(_N, _D)
    bias_col = expert_bias.reshape(_E, 1)

    out_shapes = (
        jax.ShapeDtypeStruct((2, _N), jnp.int32),
        jax.ShapeDtypeStruct((2, _N), jnp.float32),
        jax.ShapeDtypeStruct((_G, _E + 1, _LANES), jnp.float32),
    )
    idx, wts, st = pl.pallas_call(
        _router_kernel,
        grid=(_G,),
        in_specs=[
            pl.BlockSpec((_BT, _D), lambda i: (i, 0)),
            pl.BlockSpec((_D, _E), lambda i: (0, 0)),
            pl.BlockSpec((_E, 1), lambda i: (0, 0)),
        ],
        out_specs=(pl.BlockSpec((2, _BT), lambda i: (0, i)),
                   pl.BlockSpec((2, _BT), lambda i: (0, i)),
                   pl.BlockSpec((1, _E + 1, _LANES), lambda i: (i, 0, 0))),
        out_shape=out_shapes,
        compiler_params=pltpu.CompilerParams(
            dimension_semantics=("parallel",)),
    )(xf, W, bias_col)

    expert_indices = jnp.moveaxis(idx.reshape(2, _B, _S), 0, -1)
    expert_weights = jnp.moveaxis(wts.reshape(2, _B, _S), 0, -1)

    counts = jnp.sum(st[:, 0:_E, :], axis=(0, 2))
    zsum = jnp.sum(st[:, _E, :])
    z_loss = _Z_LOSS_COEFF * zsum / _N

    new_counts = expert_counts + counts
    new_total = total_tokens + jnp.float32(_N)
    current_load = new_counts / (new_total + 1e-08)
    new_expert_bias = expert_bias - _BIAS_UPDATE_SPEED * (
        current_load - 1.0 / _E)
    expert_utilization = current_load
    return (expert_indices, expert_weights, z_loss, expert_utilization,
            new_expert_bias)


# speedup vs baseline: 1.0638x; 1.0638x over previous
import jax
import jax.numpy as jnp
from jax.experimental import pallas as pl
from jax.experimental.pallas import tpu as pltpu

_B, _S, _D = 4, 8192, 768
_E, _TOPK = 8, 2
_Z_LOSS_COEFF = 1e-05
_BIAS_UPDATE_SPEED = 0.001

_BT = 4096
_N = _B * _S
_G = _N // _BT
_LANES = 128


def _router_kernel(xa_ref, xb_ref, w_ref, b_ref, idx_ref, wts_ref, st_ref):
    wt = w_ref[...].T
    _H = _D // 2
    lt = jax.lax.dot_general(
        wt[:, 0:_H], xa_ref[...], (((1,), (1,)), ((), ())),
        preferred_element_type=jnp.float32) + jax.lax.dot_general(
        wt[:, _H:_D], xb_ref[...], (((1,), (1,)), ((), ())),
        preferred_element_type=jnp.float32)
    biased = lt + b_ref[...]

    row = jax.lax.broadcasted_iota(jnp.int32, (_E, _BT), 0)
    i1 = jnp.argmax(biased, axis=0).astype(jnp.int32)
    eq1 = row == i1[None, :]
    masked = jnp.where(eq1, -jnp.inf, biased)
    i2 = jnp.argmax(masked, axis=0).astype(jnp.int32)
    eq2 = row == i2[None, :]

    l1 = jnp.sum(jnp.where(eq1, lt, 0.0), axis=0)
    l2 = jnp.sum(jnp.where(eq2, lt, 0.0), axis=0)
    t = jnp.exp(l2 - l1)
    w1v = 1.0 / (1.0 + t)

    mx = jnp.max(lt, axis=0)
    lz = mx + jnp.log(jnp.sum(jnp.exp(lt - mx[None, :]), axis=0))
    zsq = lz * lz

    c = eq1.astype(jnp.float32) + eq2.astype(jnp.float32)
    cpart = c[:, 0:_LANES]
    zpart = zsq[0:_LANES]
    for j in range(1, _BT // _LANES):
        cpart = cpart + c[:, j * _LANES:(j + 1) * _LANES]
        zpart = zpart + zsq[j * _LANES:(j + 1) * _LANES]

    idx_ref[0:1, :] = i1[None]
    idx_ref[1:2, :] = i2[None]
    wts_ref[0:1, :] = w1v[None]
    wts_ref[1:2, :] = (1.0 - w1v)[None]
    st_ref[0, 0:_E, :] = cpart
    st_ref[0, _E:_E + 1, :] = zpart[None]


@jax.jit
def kernel(x, W, expert_bias, expert_counts, total_tokens):
    xf = x.reshape(_N, _D)
    bias_col = expert_bias.reshape(_E, 1)

    out_shapes = (
        jax.ShapeDtypeStruct((2, _N), jnp.int32),
        jax.ShapeDtypeStruct((2, _N), jnp.float32),
        jax.ShapeDtypeStruct((_G, _E + 1, _LANES), jnp.float32),
    )
    idx, wts, st = pl.pallas_call(
        _router_kernel,
        grid=(_G,),
        in_specs=[
            pl.BlockSpec((_BT, _D // 2), lambda i: (i, 0)),
            pl.BlockSpec((_BT, _D // 2), lambda i: (i, 1)),
            pl.BlockSpec((_D, _E), lambda i: (0, 0)),
            pl.BlockSpec((_E, 1), lambda i: (0, 0)),
        ],
        out_specs=(pl.BlockSpec((2, _BT), lambda i: (0, i)),
                   pl.BlockSpec((2, _BT), lambda i: (0, i)),
                   pl.BlockSpec((1, _E + 1, _LANES), lambda i: (i, 0, 0))),
        out_shape=out_shapes,
        compiler_params=pltpu.CompilerParams(
            dimension_semantics=("parallel",)),
    )(xf, xf, W, bias_col)

    expert_indices = jnp.moveaxis(idx.reshape(2, _B, _S), 0, -1)
    expert_weights = jnp.moveaxis(wts.reshape(2, _B, _S), 0, -1)

    counts = jnp.sum(st[:, 0:_E, :], axis=(0, 2))
    zsum = jnp.sum(st[:, _E, :])
    z_loss = _Z_LOSS_COEFF * zsum / _N

    new_counts = expert_counts + counts
    new_total = total_tokens + jnp.float32(_N)
    current_load = new_counts / (new_total + 1e-08)
    new_expert_bias = expert_bias - _BIAS_UPDATE_SPEED * (
        current_load - 1.0 / _E)
    expert_utilization = current_load
    return (expert_indices, expert_weights, z_loss, expert_utilization,
            new_expert_bias)
